# Initial kernel scaffold; baseline (speedup 1.0000x reference)
import jax
import jax.numpy as jnp
from jax.experimental import pallas as pl
from jax.experimental.pallas import tpu as pltpu

B = 4096
IN = 16384
SPLIT = 8192
H = 1024
TAIL = 256

M_TILE = 256


def _mm_body(xd_ref, wf_ref, wr1_ref, wr2_ref, bf_ref, br_ref, out_ref):
    x = xd_ref[...].astype(jnp.bfloat16)
    xf = x[:, :SPLIT]
    xr = x[:, SPLIT:]
    acc = jnp.dot(xf, wf_ref[...], preferred_element_type=jnp.float32)
    hidden = jnp.dot(xr, wr1_ref[...], preferred_element_type=jnp.float32)
    hidden = (hidden + br_ref[...]).astype(jnp.bfloat16)
    acc += jnp.dot(hidden, wr2_ref[...], preferred_element_type=jnp.float32)
    out_ref[...] = acc + bf_ref[...]


def _fused_matmul(xd, wf, wr1, wr2, bf, br):
    grid = (B // M_TILE,)
    return pl.pallas_call(
        _mm_body,
        grid=grid,
        in_specs=[
            pl.BlockSpec((M_TILE, IN), lambda i: (i, 0)),
            pl.BlockSpec((SPLIT, H), lambda i: (0, 0)),
            pl.BlockSpec((SPLIT, TAIL), lambda i: (0, 0)),
            pl.BlockSpec((TAIL, H), lambda i: (0, 0)),
            pl.BlockSpec((1, H), lambda i: (0, 0)),
            pl.BlockSpec((1, TAIL), lambda i: (0, 0)),
        ],
        out_specs=pl.BlockSpec((M_TILE, H), lambda i: (i, 0)),
        out_shape=jax.ShapeDtypeStruct((B, H), jnp.float32),
    )(xd, wf, wr1, wr2, bf, br)


def kernel(X_values, W_freq, b_freq, W_rare_sp, b_rare_sp, W_rare_dense, X_rows, X_cols):
    xd = jnp.zeros((B, IN), jnp.float32).at[X_rows, X_cols].add(X_values)
    wf = W_freq.astype(jnp.bfloat16)
    wr1 = W_rare_sp.astype(jnp.bfloat16)
    wr2 = W_rare_dense.astype(jnp.bfloat16)
    bf = b_freq.reshape(1, H)
    br = b_rare_sp.reshape(1, TAIL)
    return _fused_matmul(xd, wf, wr1, wr2, bf, br)


# speedup vs baseline: 1.1260x; 1.1260x over previous
import jax
import jax.numpy as jnp
from jax.experimental import pallas as pl
from jax.experimental.pallas import tpu as pltpu

B = 4096
IN = 16384
SPLIT = 8192
H = 1024
TAIL = 256

M_TILE = 128


def _mm_body(xd_ref, wf_ref, wr1_ref, wr2_ref, bf_ref, br_ref, out_ref):
    x = xd_ref[...].astype(jnp.bfloat16)
    xf = x[:, :SPLIT]
    xr = x[:, SPLIT:]
    acc = jnp.dot(xf, wf_ref[...], preferred_element_type=jnp.float32)
    hidden = jnp.dot(xr, wr1_ref[...], preferred_element_type=jnp.float32)
    hidden = (hidden + br_ref[...]).astype(jnp.bfloat16)
    acc += jnp.dot(hidden, wr2_ref[...], preferred_element_type=jnp.float32)
    out_ref[...] = acc + bf_ref[...]


def _fused_matmul(xd, wf, wr1, wr2, bf, br):
    grid = (B // M_TILE,)
    return pl.pallas_call(
        _mm_body,
        grid=grid,
        in_specs=[
            pl.BlockSpec((M_TILE, IN), lambda i: (i, 0)),
            pl.BlockSpec((SPLIT, H), lambda i: (0, 0)),
            pl.BlockSpec((SPLIT, TAIL), lambda i: (0, 0)),
            pl.BlockSpec((TAIL, H), lambda i: (0, 0)),
            pl.BlockSpec((1, H), lambda i: (0, 0)),
            pl.BlockSpec((1, TAIL), lambda i: (0, 0)),
        ],
        out_specs=pl.BlockSpec((M_TILE, H), lambda i: (i, 0)),
        out_shape=jax.ShapeDtypeStruct((B, H), jnp.float32),
    )(xd, wf, wr1, wr2, bf, br)


def kernel(X_values, W_freq, b_freq, W_rare_sp, b_rare_sp, W_rare_dense, X_rows, X_cols):
    xd = jnp.zeros((B, IN), jnp.float32).at[X_rows, X_cols].add(X_values)
    wf = W_freq.astype(jnp.bfloat16)
    wr1 = W_rare_sp.astype(jnp.bfloat16)
    wr2 = W_rare_dense.astype(jnp.bfloat16)
    bf = b_freq.reshape(1, H)
    br = b_rare_sp.reshape(1, TAIL)
    return _fused_matmul(xd, wf, wr1, wr2, bf, br)
